# baseline (speedup 1.0000x reference)
import functools

import jax
import jax.numpy as jnp
from jax import lax
from jax.experimental import pallas as pl
from jax.experimental.pallas import tpu as pltpu
from jax.experimental.pallas import tpu_sc as plsc

NUM_CORES = 2
NUM_SUBCORES = 16
NW = NUM_CORES * NUM_SUBCORES

BATCH = 4096
SRC_LEN = 200
EMBED_DIM = 64
TOTAL = BATCH * SRC_LEN
BPW = TOTAL // NW
CHUNK = 512
NCHUNK = BPW // CHUNK
NPAIR = NCHUNK // 2

_mesh = plsc.VectorSubcoreMesh(
    core_axis_name="c", subcore_axis_name="s",
    num_cores=NUM_CORES, num_subcores=NUM_SUBCORES,
)


@functools.partial(
    pl.kernel,
    out_type=jax.ShapeDtypeStruct((TOTAL, EMBED_DIM), jnp.float32),
    mesh=_mesh,
    compiler_params=pltpu.CompilerParams(use_tc_tiling_on_sc=False),
    scratch_types=[
        pltpu.VMEM((BPW,), jnp.int32),
        pltpu.VMEM((CHUNK, EMBED_DIM), jnp.float32),
        pltpu.VMEM((CHUNK, EMBED_DIM), jnp.float32),
        pltpu.SemaphoreType.DMA,
        pltpu.SemaphoreType.DMA,
        pltpu.SemaphoreType.DMA,
        pltpu.SemaphoreType.DMA,
    ],
)
def _sc_gather(idx_hbm, table_hbm, out_hbm, idx_v, rows0, rows1,
               sg0, sg1, ss0, ss1):
    wid = lax.axis_index("s") * NUM_CORES + lax.axis_index("c")
    base = wid * BPW
    pltpu.sync_copy(idx_hbm.at[pl.ds(base, BPW)], idx_v)

    def gather_start(g, rows, sem):
        pltpu.async_copy(table_hbm.at[idx_v.at[pl.ds(g * CHUNK, CHUNK)]],
                         rows, sem)

    def store_start(g, rows, sem):
        pltpu.async_copy(rows, out_hbm.at[pl.ds(base + g * CHUNK, CHUNK)], sem)

    def gather_wait(rows, sem):
        pltpu.make_async_copy(table_hbm.at[idx_v.at[pl.ds(0, CHUNK)]],
                              rows, sem).wait()

    def store_wait(g, rows, sem):
        pltpu.make_async_copy(rows, out_hbm.at[pl.ds(base + g * CHUNK, CHUNK)],
                              sem).wait()

    gather_start(0, rows0, sg0)
    gather_start(1, rows1, sg1)

    @pl.loop(0, NPAIR - 1)
    def _pair(i):
        g0 = 2 * i
        g1 = g0 + 1
        gather_wait(rows0, sg0)
        store_start(g0, rows0, ss0)
        gather_wait(rows1, sg1)
        store_start(g1, rows1, ss1)
        store_wait(g0, rows0, ss0)
        gather_start(g0 + 2, rows0, sg0)
        store_wait(g1, rows1, ss1)
        gather_start(g1 + 2, rows1, sg1)

    g0 = NCHUNK - 2
    g1 = NCHUNK - 1
    gather_wait(rows0, sg0)
    store_start(g0, rows0, ss0)
    gather_wait(rows1, sg1)
    store_start(g1, rows1, ss1)
    store_wait(g0, rows0, ss0)
    store_wait(g1, rows1, ss1)


def kernel(src, table):
    idx = src.reshape(TOTAL).astype(jnp.int32)
    out = _sc_gather(idx, table)
    return out.reshape(BATCH, SRC_LEN, EMBED_DIM)


# speedup vs baseline: 1.0007x; 1.0007x over previous
import functools

import jax
import jax.numpy as jnp
from jax import lax
from jax.experimental import pallas as pl
from jax.experimental.pallas import tpu as pltpu
from jax.experimental.pallas import tpu_sc as plsc

NUM_CORES = 2
NUM_SUBCORES = 16
NW = NUM_CORES * NUM_SUBCORES

BATCH = 4096
SRC_LEN = 200
EMBED_DIM = 64
TOTAL = BATCH * SRC_LEN
BPW = TOTAL // NW
CHUNK = 400
NBUF = 4
NCHUNK = BPW // CHUNK
NGRP = NCHUNK // NBUF

_mesh = plsc.VectorSubcoreMesh(
    core_axis_name="c", subcore_axis_name="s",
    num_cores=NUM_CORES, num_subcores=NUM_SUBCORES,
)

_row_buf = pltpu.VMEM((CHUNK, EMBED_DIM), jnp.float32)


@functools.partial(
    pl.kernel,
    out_type=jax.ShapeDtypeStruct((TOTAL, EMBED_DIM), jnp.float32),
    mesh=_mesh,
    compiler_params=pltpu.CompilerParams(use_tc_tiling_on_sc=False),
    scratch_types=[
        pltpu.VMEM((BPW,), jnp.int32),
        [_row_buf] * NBUF,
        [pltpu.SemaphoreType.DMA] * NBUF,
        [pltpu.SemaphoreType.DMA] * NBUF,
    ],
)
def _sc_gather(idx_hbm, table_hbm, out_hbm, idx_v, rows, sg, ss):
    wid = lax.axis_index("s") * NUM_CORES + lax.axis_index("c")
    base = wid * BPW
    pltpu.sync_copy(idx_hbm.at[pl.ds(base, BPW)], idx_v)

    def gather_start(g, b):
        pltpu.async_copy(table_hbm.at[idx_v.at[pl.ds(g * CHUNK, CHUNK)]],
                         rows[b], sg[b])

    def store_start(g, b):
        pltpu.async_copy(rows[b], out_hbm.at[pl.ds(base + g * CHUNK, CHUNK)],
                         ss[b])

    def gather_wait(b):
        pltpu.make_async_copy(table_hbm.at[idx_v.at[pl.ds(0, CHUNK)]],
                              rows[b], sg[b]).wait()

    def store_wait(g, b):
        pltpu.make_async_copy(rows[b], out_hbm.at[pl.ds(base + g * CHUNK,
                                                        CHUNK)],
                              ss[b]).wait()

    for b in range(NBUF):
        gather_start(b, b)

    @pl.loop(0, NGRP - 1)
    def _round(i):
        g0 = i * NBUF
        for b in range(NBUF):
            gather_wait(b)
            store_start(g0 + b, b)
        for b in range(NBUF):
            store_wait(g0 + b, b)
            gather_start(g0 + NBUF + b, b)

    g0 = NCHUNK - NBUF
    for b in range(NBUF):
        gather_wait(b)
        store_start(g0 + b, b)
    for b in range(NBUF):
        store_wait(g0 + b, b)


def kernel(src, table):
    idx = src.reshape(TOTAL).astype(jnp.int32)
    out = _sc_gather(idx, table)
    return out.reshape(BATCH, SRC_LEN, EMBED_DIM)
